# baseline (speedup 1.0000x reference)
import dataclasses
import functools

import numpy as np
import jax
import jax.numpy as jnp
from jax import lax
from jax.experimental import pallas as pl
from jax.experimental.pallas import tpu as pltpu
from jax.experimental.pallas import tpu_sc as plsc

_BOUND = 1.0
_NUM_LEVELS = 16
_BASE_RES = 16
_LOG2_HASH = 19
_MAX_RES = 2048
_N = 262144
_P1 = 2654435761
_P2 = 805459861
_IN_DIM = 2 * _NUM_LEVELS


def _level_meta():
    g = np.exp((np.log(_MAX_RES) - np.log(_BASE_RES)) / (_NUM_LEVELS - 1))
    levels, off = [], 0
    for l in range(_NUM_LEVELS):
        res = int(np.floor(_BASE_RES * (g ** l)))
        size = min((res + 1) ** 3, 2 ** _LOG2_HASH)
        size = int(np.ceil(size / 8) * 8)
        dense = (res + 1) ** 3 <= size
        levels.append((res, size, off, dense))
        off += size
    return levels, off


_LEVELS, _TOTAL_ROWS = _level_meta()
for _res, _size, _off, _dense in _LEVELS:
    assert _dense or (_size & (_size - 1)) == 0
assert _TOTAL_ROWS % 4 == 0

_NC, _NS = 2, 16
_NW = _NC * _NS
_PER_W = _N // _NW
_C = 512
_NIDX = 8 * _C


def _encode(xs, ys, zs, table):
    mesh = plsc.VectorSubcoreMesh(
        core_axis_name="c", subcore_axis_name="s", num_cores=_NC, num_subcores=_NS
    )
    cp = pltpu.CompilerParams()
    if "needs_layout_passes" in pltpu.CompilerParams.__dataclass_fields__:
        cp = dataclasses.replace(cp, needs_layout_passes=False)
    if "use_tc_tiling_on_sc" in pltpu.CompilerParams.__dataclass_fields__:
        cp = dataclasses.replace(cp, use_tc_tiling_on_sc=False)

    @functools.partial(
        pl.kernel,
        compiler_params=cp,
        out_type=jax.ShapeDtypeStruct((_N * _IN_DIM,), jnp.float32),
        mesh=mesh,
        scratch_types=[
            pltpu.VMEM((_PER_W,), jnp.float32),
            pltpu.VMEM((_PER_W,), jnp.float32),
            pltpu.VMEM((_PER_W,), jnp.float32),
            pltpu.VMEM((_NIDX,), jnp.int32),
            pltpu.VMEM((_NIDX,), jnp.int32),
            pltpu.VMEM((_NIDX,), jnp.float32),
            pltpu.VMEM((_NIDX,), jnp.float32),
            pltpu.VMEM((_C * _IN_DIM,), jnp.float32),
            pltpu.SemaphoreType.DMA,
        ],
    )
    def enc(x_hbm, y_hbm, z_hbm, tab_hbm, out_hbm, x_v, y_v, z_v, idxe_v, idxo_v, val0_v, val1_v, feat_v, sem):
        wid = lax.axis_index("c") * _NS + lax.axis_index("s")
        wbase = wid * _PER_W
        pltpu.sync_copy(x_hbm.at[pl.ds(wbase, _PER_W)], x_v)
        pltpu.sync_copy(y_hbm.at[pl.ds(wbase, _PER_W)], y_v)
        pltpu.sync_copy(z_hbm.at[pl.ds(wbase, _PER_W)], z_v)
        iota = lax.iota(jnp.int32, 16)
        iota32 = iota * _IN_DIM

        def norm01(v):
            return jnp.minimum(jnp.maximum((v + _BOUND) * (0.5 / _BOUND), 0.0), 1.0)

        @pl.loop(0, _PER_W, step=_C)
        def _chunk(cb):
            for l, (res, size, off, dense) in enumerate(_LEVELS):
                scale = float(res - 1)

                @pl.loop(0, _C, step=16)
                def _idx_pass(po):
                    xb = cb + po
                    x0 = (norm01(x_v[pl.ds(xb, 16)]) * scale).astype(jnp.int32)
                    y0 = (norm01(y_v[pl.ds(xb, 16)]) * scale).astype(jnp.int32)
                    z0 = (norm01(z_v[pl.ds(xb, 16)]) * scale).astype(jnp.int32)
                    if dense:
                        s1, s2 = res + 1, (res + 1) * (res + 1)
                        xs = (x0, x0 + 1)
                        ys = (y0 * s1, (y0 + 1) * s1)
                        zs = (z0 * s2 + off, (z0 + 1) * s2 + off)
                        for c in range(8):
                            ix = xs[c & 1] + ys[(c >> 1) & 1] + zs[(c >> 2) & 1]
                            ie = ix * 2
                            idxe_v[pl.ds(c * _C + po, 16)] = ie
                            idxo_v[pl.ds(c * _C + po, 16)] = ie + 1
                    else:
                        msk = jnp.uint32(size - 1)
                        x0u = x0.astype(jnp.uint32)
                        y0u = y0.astype(jnp.uint32)
                        z0u = z0.astype(jnp.uint32)
                        xs = (x0u, x0u + jnp.uint32(1))
                        ys = (y0u * jnp.uint32(_P1), (y0u + jnp.uint32(1)) * jnp.uint32(_P1))
                        zs = (z0u * jnp.uint32(_P2), (z0u + jnp.uint32(1)) * jnp.uint32(_P2))
                        for c in range(8):
                            h = xs[c & 1] ^ ys[(c >> 1) & 1] ^ zs[(c >> 2) & 1]
                            ix = (h & msk).astype(jnp.int32) + off
                            ie = ix * 2
                            idxe_v[pl.ds(c * _C + po, 16)] = ie
                            idxo_v[pl.ds(c * _C + po, 16)] = ie + 1

                cp0 = pltpu.async_copy(tab_hbm.at[idxe_v], val0_v, sem)
                cp1 = pltpu.async_copy(tab_hbm.at[idxo_v], val1_v, sem)
                cp0.wait()
                cp1.wait()

                @pl.loop(0, _C, step=16)
                def _acc_pass(po):
                    xb = cb + po
                    px = norm01(x_v[pl.ds(xb, 16)]) * scale
                    py = norm01(y_v[pl.ds(xb, 16)]) * scale
                    pz = norm01(z_v[pl.ds(xb, 16)]) * scale
                    fx = px - px.astype(jnp.int32).astype(jnp.float32)
                    fy = py - py.astype(jnp.int32).astype(jnp.float32)
                    fz = pz - pz.astype(jnp.int32).astype(jnp.float32)
                    wx = (1.0 - fx, fx)
                    wy = (1.0 - fy, fy)
                    wz = (1.0 - fz, fz)
                    wxy = [wx[i & 1] * wy[i >> 1] for i in range(4)]
                    f0 = jnp.zeros((16,), jnp.float32)
                    f1 = jnp.zeros((16,), jnp.float32)
                    for c in range(8):
                        w = wxy[c & 3] * wz[(c >> 2) & 1]
                        f0 = f0 + w * val0_v[pl.ds(c * _C + po, 16)]
                        f1 = f1 + w * val1_v[pl.ds(c * _C + po, 16)]
                    pp32 = po * _IN_DIM + iota32
                    plsc.store_scatter(feat_v, [pp32 + (2 * l)], f0)
                    plsc.store_scatter(feat_v, [pp32 + (2 * l + 1)], f1)

            pltpu.sync_copy(feat_v, out_hbm.at[pl.ds((wbase + cb) * _IN_DIM, _C * _IN_DIM)])

    return enc(xs, ys, zs, table)


def _mlp(feat, W0, b0, W1, b1, Wout, bout):
    B = 4096

    def body(x_ref, w0, b0r, w1, b1r, wo, bor, sig_ref, geo_ref):
        x = x_ref[...]
        h = jnp.maximum(jnp.dot(x, w0[...], preferred_element_type=jnp.float32) + b0r[...], 0.0)
        h = jnp.maximum(jnp.dot(h, w1[...], preferred_element_type=jnp.float32) + b1r[...], 0.0)
        o = jnp.dot(h, wo[...], preferred_element_type=jnp.float32) + bor[...]
        sig_ref[...] = jnp.exp(jnp.clip(o[:, :1], -15.0, 15.0))
        geo_ref[...] = o[:, 1:]

    sig, geo = pl.pallas_call(
        body,
        grid=(_N // B,),
        in_specs=[
            pl.BlockSpec((B, _IN_DIM), lambda i: (i, 0)),
            pl.BlockSpec((_IN_DIM, 64), lambda i: (0, 0)),
            pl.BlockSpec((1, 64), lambda i: (0, 0)),
            pl.BlockSpec((64, 64), lambda i: (0, 0)),
            pl.BlockSpec((1, 64), lambda i: (0, 0)),
            pl.BlockSpec((64, 16), lambda i: (0, 0)),
            pl.BlockSpec((1, 16), lambda i: (0, 0)),
        ],
        out_specs=[
            pl.BlockSpec((B, 1), lambda i: (i, 0)),
            pl.BlockSpec((B, 15), lambda i: (i, 0)),
        ],
        out_shape=[
            jax.ShapeDtypeStruct((_N, 1), jnp.float32),
            jax.ShapeDtypeStruct((_N, 15), jnp.float32),
        ],
    )(feat, W0, b0.reshape(1, -1), W1, b1.reshape(1, -1), Wout, bout.reshape(1, -1))
    return sig.reshape(-1), geo


def kernel(xyzs, table, W0, b0, W1, b1, Wout, bout):
    feat = _encode(xyzs[:, 0], xyzs[:, 1], xyzs[:, 2], table.reshape(-1))
    feat = feat.reshape(_N, _IN_DIM)
    return _mlp(feat, W0, b0, W1, b1, Wout, bout)


# speedup vs baseline: 3.1489x; 3.1489x over previous
import dataclasses
import functools

import numpy as np
import jax
import jax.numpy as jnp
from jax import lax
from jax.experimental import pallas as pl
from jax.experimental.pallas import tpu as pltpu
from jax.experimental.pallas import tpu_sc as plsc

_BOUND = 1.0
_NUM_LEVELS = 16
_BASE_RES = 16
_LOG2_HASH = 19
_MAX_RES = 2048
_N = 262144
_P1 = 2654435761
_P2 = 805459861
_IN_DIM = 2 * _NUM_LEVELS


def _level_meta():
    g = np.exp((np.log(_MAX_RES) - np.log(_BASE_RES)) / (_NUM_LEVELS - 1))
    levels, off = [], 0
    for l in range(_NUM_LEVELS):
        res = int(np.floor(_BASE_RES * (g ** l)))
        size = min((res + 1) ** 3, 2 ** _LOG2_HASH)
        size = int(np.ceil(size / 8) * 8)
        dense = (res + 1) ** 3 <= size
        levels.append((res, size, off, dense))
        off += size
    return levels, off


_LEVELS, _TOTAL_ROWS = _level_meta()
for _res, _size, _off, _dense in _LEVELS:
    assert _dense or (_size & (_size - 1)) == 0
assert _TOTAL_ROWS % 4 == 0

_NC, _NS = 2, 16
_NW = _NC * _NS
_PER_W = _N // _NW
_C = 512
_NIDX = 8 * _C


def _encode(xs, ys, zs, tab0, tab1):
    mesh = plsc.VectorSubcoreMesh(
        core_axis_name="c", subcore_axis_name="s", num_cores=_NC, num_subcores=_NS
    )
    cp = pltpu.CompilerParams()
    if "needs_layout_passes" in pltpu.CompilerParams.__dataclass_fields__:
        cp = dataclasses.replace(cp, needs_layout_passes=False)
    if "use_tc_tiling_on_sc" in pltpu.CompilerParams.__dataclass_fields__:
        cp = dataclasses.replace(cp, use_tc_tiling_on_sc=False)

    @functools.partial(
        pl.kernel,
        compiler_params=cp,
        out_type=jax.ShapeDtypeStruct((_IN_DIM, _N), jnp.float32),
        mesh=mesh,
        scratch_types=[
            pltpu.VMEM((_PER_W,), jnp.float32),
            pltpu.VMEM((_PER_W,), jnp.float32),
            pltpu.VMEM((_PER_W,), jnp.float32),
            pltpu.VMEM((_NIDX,), jnp.int32),
            pltpu.VMEM((_NIDX,), jnp.float32),
            pltpu.VMEM((_NIDX,), jnp.float32),
            pltpu.VMEM((_IN_DIM, _C), jnp.float32),
            pltpu.SemaphoreType.DMA,
        ],
    )
    def enc(x_hbm, y_hbm, z_hbm, tab0_hbm, tab1_hbm, out_hbm, x_v, y_v, z_v, idx_v, val0_v, val1_v, feat_v, sem):
        wid = lax.axis_index("c") * _NS + lax.axis_index("s")
        wbase = wid * _PER_W
        pltpu.sync_copy(x_hbm.at[pl.ds(wbase, _PER_W)], x_v)
        pltpu.sync_copy(y_hbm.at[pl.ds(wbase, _PER_W)], y_v)
        pltpu.sync_copy(z_hbm.at[pl.ds(wbase, _PER_W)], z_v)

        def norm01(v):
            return jnp.minimum(jnp.maximum((v + _BOUND) * (0.5 / _BOUND), 0.0), 1.0)

        @pl.loop(0, _PER_W, step=_C)
        def _chunk(cb):
            for l, (res, size, off, dense) in enumerate(_LEVELS):
                scale = float(res - 1)

                @pl.loop(0, _C, step=16)
                def _idx_pass(po):
                    xb = cb + po
                    x0 = (norm01(x_v[pl.ds(xb, 16)]) * scale).astype(jnp.int32)
                    y0 = (norm01(y_v[pl.ds(xb, 16)]) * scale).astype(jnp.int32)
                    z0 = (norm01(z_v[pl.ds(xb, 16)]) * scale).astype(jnp.int32)
                    if dense:
                        s1, s2 = res + 1, (res + 1) * (res + 1)
                        xs = (x0, x0 + 1)
                        ys = (y0 * s1, (y0 + 1) * s1)
                        zs = (z0 * s2 + off, (z0 + 1) * s2 + off)
                        for c in range(8):
                            ix = xs[c & 1] + ys[(c >> 1) & 1] + zs[(c >> 2) & 1]
                            idx_v[pl.ds(c * _C + po, 16)] = ix
                    else:
                        msk = jnp.uint32(size - 1)
                        x0u = x0.astype(jnp.uint32)
                        y0u = y0.astype(jnp.uint32)
                        z0u = z0.astype(jnp.uint32)
                        xs = (x0u, x0u + jnp.uint32(1))
                        ys = (y0u * jnp.uint32(_P1), (y0u + jnp.uint32(1)) * jnp.uint32(_P1))
                        zs = (z0u * jnp.uint32(_P2), (z0u + jnp.uint32(1)) * jnp.uint32(_P2))
                        for c in range(8):
                            h = xs[c & 1] ^ ys[(c >> 1) & 1] ^ zs[(c >> 2) & 1]
                            ix = (h & msk).astype(jnp.int32) + off
                            idx_v[pl.ds(c * _C + po, 16)] = ix

                cp0 = pltpu.async_copy(tab0_hbm.at[idx_v], val0_v, sem)
                cp1 = pltpu.async_copy(tab1_hbm.at[idx_v], val1_v, sem)
                cp0.wait()
                cp1.wait()

                @pl.loop(0, _C, step=16)
                def _acc_pass(po):
                    xb = cb + po
                    px = norm01(x_v[pl.ds(xb, 16)]) * scale
                    py = norm01(y_v[pl.ds(xb, 16)]) * scale
                    pz = norm01(z_v[pl.ds(xb, 16)]) * scale
                    fx = px - px.astype(jnp.int32).astype(jnp.float32)
                    fy = py - py.astype(jnp.int32).astype(jnp.float32)
                    fz = pz - pz.astype(jnp.int32).astype(jnp.float32)
                    wx = (1.0 - fx, fx)
                    wy = (1.0 - fy, fy)
                    wz = (1.0 - fz, fz)
                    wxy = [wx[i & 1] * wy[i >> 1] for i in range(4)]
                    f0 = jnp.zeros((16,), jnp.float32)
                    f1 = jnp.zeros((16,), jnp.float32)
                    for c in range(8):
                        w = wxy[c & 3] * wz[(c >> 2) & 1]
                        f0 = f0 + w * val0_v[pl.ds(c * _C + po, 16)]
                        f1 = f1 + w * val1_v[pl.ds(c * _C + po, 16)]
                    feat_v[2 * l, pl.ds(po, 16)] = f0
                    feat_v[2 * l + 1, pl.ds(po, 16)] = f1

            pltpu.sync_copy(feat_v, out_hbm.at[:, pl.ds(wbase + cb, _C)])

    return enc(xs, ys, zs, tab0, tab1)


def _mlp(feat_t, W0, b0, W1, b1, Wout, bout):
    B = 4096

    def body(x_ref, w0, b0r, w1, b1r, wo, bor, sig_ref, geo_ref):
        x = x_ref[...]
        h = jax.lax.dot_general(w0[...], x, (((0,), (0,)), ((), ())),
                                preferred_element_type=jnp.float32)
        h = jnp.maximum(h + b0r[...], 0.0)
        h = jax.lax.dot_general(w1[...], h, (((0,), (0,)), ((), ())),
                                preferred_element_type=jnp.float32)
        h = jnp.maximum(h + b1r[...], 0.0)
        o = jax.lax.dot_general(wo[...], h, (((0,), (0,)), ((), ())),
                                preferred_element_type=jnp.float32)
        o = o + bor[...]
        sig_ref[...] = jnp.exp(jnp.clip(o[:1, :], -15.0, 15.0))
        geo_ref[...] = o[1:, :]

    sig, geo_t = pl.pallas_call(
        body,
        grid=(_N // B,),
        in_specs=[
            pl.BlockSpec((_IN_DIM, B), lambda i: (0, i)),
            pl.BlockSpec((_IN_DIM, 64), lambda i: (0, 0)),
            pl.BlockSpec((64, 1), lambda i: (0, 0)),
            pl.BlockSpec((64, 64), lambda i: (0, 0)),
            pl.BlockSpec((64, 1), lambda i: (0, 0)),
            pl.BlockSpec((64, 16), lambda i: (0, 0)),
            pl.BlockSpec((16, 1), lambda i: (0, 0)),
        ],
        out_specs=[
            pl.BlockSpec((1, B), lambda i: (0, i)),
            pl.BlockSpec((15, B), lambda i: (0, i)),
        ],
        out_shape=[
            jax.ShapeDtypeStruct((1, _N), jnp.float32),
            jax.ShapeDtypeStruct((15, _N), jnp.float32),
        ],
    )(feat_t, W0, b0.reshape(-1, 1), W1, b1.reshape(-1, 1), Wout, bout.reshape(-1, 1))
    return sig.reshape(-1), geo_t.T


def kernel(xyzs, table, W0, b0, W1, b1, Wout, bout):
    feat_t = _encode(xyzs[:, 0], xyzs[:, 1], xyzs[:, 2], table[:, 0], table[:, 1])
    return _mlp(feat_t, W0, b0, W1, b1, Wout, bout)


# speedup vs baseline: 3.4197x; 1.0860x over previous
import dataclasses
import functools

import numpy as np
import jax
import jax.numpy as jnp
from jax import lax
from jax.experimental import pallas as pl
from jax.experimental.pallas import tpu as pltpu
from jax.experimental.pallas import tpu_sc as plsc

_BOUND = 1.0
_NUM_LEVELS = 16
_BASE_RES = 16
_LOG2_HASH = 19
_MAX_RES = 2048
_N = 262144
_P1 = 2654435761
_P2 = 805459861
_IN_DIM = 2 * _NUM_LEVELS


def _level_meta():
    g = np.exp((np.log(_MAX_RES) - np.log(_BASE_RES)) / (_NUM_LEVELS - 1))
    levels, off = [], 0
    for l in range(_NUM_LEVELS):
        res = int(np.floor(_BASE_RES * (g ** l)))
        size = min((res + 1) ** 3, 2 ** _LOG2_HASH)
        size = int(np.ceil(size / 8) * 8)
        dense = (res + 1) ** 3 <= size
        levels.append((res, size, off, dense))
        off += size
    return levels, off


_LEVELS, _TOTAL_ROWS = _level_meta()
for _res, _size, _off, _dense in _LEVELS:
    assert _dense or (_size & (_size - 1)) == 0
assert _TOTAL_ROWS % 4 == 0

_NC, _NS = 2, 16
_NW = _NC * _NS
_PER_W = _N // _NW
_C = 512
_NIDX = 8 * _C


def _encode(xs, ys, zs, tab0, tab1):
    mesh = plsc.VectorSubcoreMesh(
        core_axis_name="c", subcore_axis_name="s", num_cores=_NC, num_subcores=_NS
    )
    cp = pltpu.CompilerParams()
    if "needs_layout_passes" in pltpu.CompilerParams.__dataclass_fields__:
        cp = dataclasses.replace(cp, needs_layout_passes=False)
    if "use_tc_tiling_on_sc" in pltpu.CompilerParams.__dataclass_fields__:
        cp = dataclasses.replace(cp, use_tc_tiling_on_sc=False)

    @functools.partial(
        pl.kernel,
        compiler_params=cp,
        out_type=jax.ShapeDtypeStruct((_IN_DIM, _N), jnp.float32),
        mesh=mesh,
        scratch_types=[
            pltpu.VMEM((_PER_W,), jnp.float32),
            pltpu.VMEM((_PER_W,), jnp.float32),
            pltpu.VMEM((_PER_W,), jnp.float32),
            pltpu.VMEM((_NIDX,), jnp.int32),
            pltpu.VMEM((_NIDX,), jnp.int32),
            pltpu.VMEM((_NIDX,), jnp.float32),
            pltpu.VMEM((_NIDX,), jnp.float32),
            pltpu.VMEM((_NIDX,), jnp.float32),
            pltpu.VMEM((_NIDX,), jnp.float32),
            pltpu.VMEM((_IN_DIM, _C), jnp.float32),
            pltpu.SemaphoreType.DMA,
            pltpu.SemaphoreType.DMA,
        ],
    )
    def enc(x_hbm, y_hbm, z_hbm, tab0_hbm, tab1_hbm, out_hbm, x_v, y_v, z_v,
            idx_a, idx_b, val0_a, val0_b, val1_a, val1_b, feat_v, sem_a, sem_b):
        wid = lax.axis_index("c") * _NS + lax.axis_index("s")
        wbase = wid * _PER_W
        pltpu.sync_copy(x_hbm.at[pl.ds(wbase, _PER_W)], x_v)
        pltpu.sync_copy(y_hbm.at[pl.ds(wbase, _PER_W)], y_v)
        pltpu.sync_copy(z_hbm.at[pl.ds(wbase, _PER_W)], z_v)

        def norm01(v):
            return jnp.minimum(jnp.maximum((v + _BOUND) * (0.5 / _BOUND), 0.0), 1.0)

        def idx_pass(l, cb, idx_v):
            res, size, off, dense = _LEVELS[l]
            scale = float(res - 1)

            @pl.loop(0, _C, step=16)
            def _idx(po):
                xb = cb + po
                x0 = (norm01(x_v[pl.ds(xb, 16)]) * scale).astype(jnp.int32)
                y0 = (norm01(y_v[pl.ds(xb, 16)]) * scale).astype(jnp.int32)
                z0 = (norm01(z_v[pl.ds(xb, 16)]) * scale).astype(jnp.int32)
                if dense:
                    s1, s2 = res + 1, (res + 1) * (res + 1)
                    xs = (x0, x0 + 1)
                    ys = (y0 * s1, (y0 + 1) * s1)
                    zs = (z0 * s2 + off, (z0 + 1) * s2 + off)
                    for c in range(8):
                        ix = xs[c & 1] + ys[(c >> 1) & 1] + zs[(c >> 2) & 1]
                        idx_v[pl.ds(c * _C + po, 16)] = ix
                else:
                    msk = jnp.uint32(size - 1)
                    x0u = x0.astype(jnp.uint32)
                    y0u = y0.astype(jnp.uint32)
                    z0u = z0.astype(jnp.uint32)
                    xs = (x0u, x0u + jnp.uint32(1))
                    ys = (y0u * jnp.uint32(_P1), (y0u + jnp.uint32(1)) * jnp.uint32(_P1))
                    zs = (z0u * jnp.uint32(_P2), (z0u + jnp.uint32(1)) * jnp.uint32(_P2))
                    for c in range(8):
                        h = xs[c & 1] ^ ys[(c >> 1) & 1] ^ zs[(c >> 2) & 1]
                        ix = (h & msk).astype(jnp.int32) + off
                        idx_v[pl.ds(c * _C + po, 16)] = ix

        def acc_pass(l, cb, val0_v, val1_v):
            res = _LEVELS[l][0]
            scale = float(res - 1)

            @pl.loop(0, _C, step=16)
            def _acc(po):
                xb = cb + po
                px = norm01(x_v[pl.ds(xb, 16)]) * scale
                py = norm01(y_v[pl.ds(xb, 16)]) * scale
                pz = norm01(z_v[pl.ds(xb, 16)]) * scale
                fx = px - px.astype(jnp.int32).astype(jnp.float32)
                fy = py - py.astype(jnp.int32).astype(jnp.float32)
                fz = pz - pz.astype(jnp.int32).astype(jnp.float32)
                wx = (1.0 - fx, fx)
                wy = (1.0 - fy, fy)
                wz = (1.0 - fz, fz)
                wxy = [wx[i & 1] * wy[i >> 1] for i in range(4)]
                f0 = jnp.zeros((16,), jnp.float32)
                f1 = jnp.zeros((16,), jnp.float32)
                for c in range(8):
                    w = wxy[c & 3] * wz[(c >> 2) & 1]
                    f0 = f0 + w * val0_v[pl.ds(c * _C + po, 16)]
                    f1 = f1 + w * val1_v[pl.ds(c * _C + po, 16)]
                feat_v[2 * l, pl.ds(po, 16)] = f0
                feat_v[2 * l + 1, pl.ds(po, 16)] = f1

        bufs = ((idx_a, val0_a, val1_a, sem_a), (idx_b, val0_b, val1_b, sem_b))

        def start(l, cb):
            idx_v, v0, v1, sem = bufs[l % 2]
            idx_pass(l, cb, idx_v)
            c0 = pltpu.async_copy(tab0_hbm.at[idx_v], v0, sem)
            c1 = pltpu.async_copy(tab1_hbm.at[idx_v], v1, sem)
            return (c0, c1)

        @pl.loop(0, _PER_W, step=_C)
        def _chunk(cb):
            cps = start(0, cb)
            for l in range(1, _NUM_LEVELS):
                nxt = start(l, cb)
                cps[0].wait()
                cps[1].wait()
                acc_pass(l - 1, cb, bufs[(l - 1) % 2][1], bufs[(l - 1) % 2][2])
                cps = nxt
            cps[0].wait()
            cps[1].wait()
            acc_pass(_NUM_LEVELS - 1, cb, bufs[(_NUM_LEVELS - 1) % 2][1],
                     bufs[(_NUM_LEVELS - 1) % 2][2])

            pltpu.sync_copy(feat_v, out_hbm.at[:, pl.ds(wbase + cb, _C)])

    return enc(xs, ys, zs, tab0, tab1)


def _mlp(feat_t, W0, b0, W1, b1, Wout, bout):
    B = 4096

    def body(x_ref, w0, b0r, w1, b1r, wo, bor, sig_ref, geo_ref):
        x = x_ref[...]
        h = jax.lax.dot_general(w0[...], x, (((0,), (0,)), ((), ())),
                                preferred_element_type=jnp.float32)
        h = jnp.maximum(h + b0r[...], 0.0)
        h = jax.lax.dot_general(w1[...], h, (((0,), (0,)), ((), ())),
                                preferred_element_type=jnp.float32)
        h = jnp.maximum(h + b1r[...], 0.0)
        o = jax.lax.dot_general(wo[...], h, (((0,), (0,)), ((), ())),
                                preferred_element_type=jnp.float32)
        o = o + bor[...]
        sig_ref[...] = jnp.exp(jnp.clip(o[:1, :], -15.0, 15.0))
        geo_ref[...] = o[1:, :]

    sig, geo_t = pl.pallas_call(
        body,
        grid=(_N // B,),
        in_specs=[
            pl.BlockSpec((_IN_DIM, B), lambda i: (0, i)),
            pl.BlockSpec((_IN_DIM, 64), lambda i: (0, 0)),
            pl.BlockSpec((64, 1), lambda i: (0, 0)),
            pl.BlockSpec((64, 64), lambda i: (0, 0)),
            pl.BlockSpec((64, 1), lambda i: (0, 0)),
            pl.BlockSpec((64, 16), lambda i: (0, 0)),
            pl.BlockSpec((16, 1), lambda i: (0, 0)),
        ],
        out_specs=[
            pl.BlockSpec((1, B), lambda i: (0, i)),
            pl.BlockSpec((15, B), lambda i: (0, i)),
        ],
        out_shape=[
            jax.ShapeDtypeStruct((1, _N), jnp.float32),
            jax.ShapeDtypeStruct((15, _N), jnp.float32),
        ],
    )(feat_t, W0, b0.reshape(-1, 1), W1, b1.reshape(-1, 1), Wout, bout.reshape(-1, 1))
    return sig.reshape(-1), geo_t.T


def kernel(xyzs, table, W0, b0, W1, b1, Wout, bout):
    feat_t = _encode(xyzs[:, 0], xyzs[:, 1], xyzs[:, 2], table[:, 0], table[:, 1])
    return _mlp(feat_t, W0, b0, W1, b1, Wout, bout)


# speedup vs baseline: 6.1888x; 1.8098x over previous
import dataclasses
import functools

import numpy as np
import jax
import jax.numpy as jnp
from jax import lax
from jax.experimental import pallas as pl
from jax.experimental.pallas import tpu as pltpu
from jax.experimental.pallas import tpu_sc as plsc

_BOUND = 1.0
_NUM_LEVELS = 16
_BASE_RES = 16
_LOG2_HASH = 19
_MAX_RES = 2048
_N = 262144
_P1 = 2654435761
_P2 = 805459861
_IN_DIM = 2 * _NUM_LEVELS


def _level_meta():
    g = np.exp((np.log(_MAX_RES) - np.log(_BASE_RES)) / (_NUM_LEVELS - 1))
    levels, off = [], 0
    for l in range(_NUM_LEVELS):
        res = int(np.floor(_BASE_RES * (g ** l)))
        size = min((res + 1) ** 3, 2 ** _LOG2_HASH)
        size = int(np.ceil(size / 8) * 8)
        dense = (res + 1) ** 3 <= size
        levels.append((res, size, off, dense))
        off += size
    return levels, off


_LEVELS, _TOTAL_ROWS = _level_meta()
for _res, _size, _off, _dense in _LEVELS:
    assert _dense or (_size & (_size - 1)) == 0
assert _TOTAL_ROWS % 4 == 0

_SH_ROWS = next(o for (_, _, o, d) in _LEVELS if not d)
_NC, _NS = 2, 16
_NW = _NC * _NS
_PER_W = _N // _NW
_C = 512
_NIDX = 8 * _C


def _encode(xs, ys, zs, tab0, tab1):
    mesh = plsc.VectorSubcoreMesh(
        core_axis_name="c", subcore_axis_name="s", num_cores=_NC, num_subcores=_NS
    )
    cp = pltpu.CompilerParams()
    if "needs_layout_passes" in pltpu.CompilerParams.__dataclass_fields__:
        cp = dataclasses.replace(cp, needs_layout_passes=False)
    if "use_tc_tiling_on_sc" in pltpu.CompilerParams.__dataclass_fields__:
        cp = dataclasses.replace(cp, use_tc_tiling_on_sc=False)

    @functools.partial(
        pl.kernel,
        compiler_params=cp,
        out_type=jax.ShapeDtypeStruct((_IN_DIM, _N), jnp.float32),
        mesh=mesh,
        scratch_types=[
            pltpu.VMEM((_PER_W,), jnp.float32),
            pltpu.VMEM((_PER_W,), jnp.float32),
            pltpu.VMEM((_PER_W,), jnp.float32),
            pltpu.VMEM((_NIDX,), jnp.int32),
            pltpu.VMEM((_NIDX,), jnp.int32),
            pltpu.VMEM((_NIDX,), jnp.float32),
            pltpu.VMEM((_NIDX,), jnp.float32),
            pltpu.VMEM((_NIDX,), jnp.float32),
            pltpu.VMEM((_NIDX,), jnp.float32),
            pltpu.VMEM((_IN_DIM, _C), jnp.float32),
            pltpu.VMEM_SHARED((_SH_ROWS,), jnp.float32),
            pltpu.VMEM_SHARED((_SH_ROWS,), jnp.float32),
            pltpu.SemaphoreType.DMA,
            pltpu.SemaphoreType.DMA,
        ],
    )
    def enc(x_hbm, y_hbm, z_hbm, tab0_hbm, tab1_hbm, out_hbm, x_v, y_v, z_v,
            idx_a, idx_b, val0_a, val0_b, val1_a, val1_b, feat_v, sh0, sh1, sem_a, sem_b):
        wid = lax.axis_index("c") * _NS + lax.axis_index("s")
        wbase = wid * _PER_W

        @pl.when(lax.axis_index("s") == 0)
        def _stage():
            pltpu.sync_copy(tab0_hbm.at[pl.ds(0, _SH_ROWS)], sh0)
            pltpu.sync_copy(tab1_hbm.at[pl.ds(0, _SH_ROWS)], sh1)

        pltpu.sync_copy(x_hbm.at[pl.ds(wbase, _PER_W)], x_v)
        pltpu.sync_copy(y_hbm.at[pl.ds(wbase, _PER_W)], y_v)
        pltpu.sync_copy(z_hbm.at[pl.ds(wbase, _PER_W)], z_v)
        plsc.subcore_barrier()

        def norm01(v):
            return jnp.minimum(jnp.maximum((v + _BOUND) * (0.5 / _BOUND), 0.0), 1.0)

        def idx_pass(l, cb, idx_v):
            res, size, off, dense = _LEVELS[l]
            scale = float(res - 1)

            @pl.loop(0, _C, step=16)
            def _idx(po):
                xb = cb + po
                x0 = (norm01(x_v[pl.ds(xb, 16)]) * scale).astype(jnp.int32)
                y0 = (norm01(y_v[pl.ds(xb, 16)]) * scale).astype(jnp.int32)
                z0 = (norm01(z_v[pl.ds(xb, 16)]) * scale).astype(jnp.int32)
                if dense:
                    s1, s2 = res + 1, (res + 1) * (res + 1)
                    xs = (x0, x0 + 1)
                    ys = (y0 * s1, (y0 + 1) * s1)
                    zs = (z0 * s2 + off, (z0 + 1) * s2 + off)
                    for c in range(8):
                        ix = xs[c & 1] + ys[(c >> 1) & 1] + zs[(c >> 2) & 1]
                        idx_v[pl.ds(c * _C + po, 16)] = ix
                else:
                    msk = jnp.uint32(size - 1)
                    x0u = x0.astype(jnp.uint32)
                    y0u = y0.astype(jnp.uint32)
                    z0u = z0.astype(jnp.uint32)
                    xs = (x0u, x0u + jnp.uint32(1))
                    ys = (y0u * jnp.uint32(_P1), (y0u + jnp.uint32(1)) * jnp.uint32(_P1))
                    zs = (z0u * jnp.uint32(_P2), (z0u + jnp.uint32(1)) * jnp.uint32(_P2))
                    for c in range(8):
                        h = xs[c & 1] ^ ys[(c >> 1) & 1] ^ zs[(c >> 2) & 1]
                        ix = (h & msk).astype(jnp.int32) + off
                        idx_v[pl.ds(c * _C + po, 16)] = ix

        def acc_pass(l, cb, val0_v, val1_v):
            res = _LEVELS[l][0]
            scale = float(res - 1)

            @pl.loop(0, _C, step=16)
            def _acc(po):
                xb = cb + po
                px = norm01(x_v[pl.ds(xb, 16)]) * scale
                py = norm01(y_v[pl.ds(xb, 16)]) * scale
                pz = norm01(z_v[pl.ds(xb, 16)]) * scale
                fx = px - px.astype(jnp.int32).astype(jnp.float32)
                fy = py - py.astype(jnp.int32).astype(jnp.float32)
                fz = pz - pz.astype(jnp.int32).astype(jnp.float32)
                wx = (1.0 - fx, fx)
                wy = (1.0 - fy, fy)
                wz = (1.0 - fz, fz)
                wxy = [wx[i & 1] * wy[i >> 1] for i in range(4)]
                f0 = jnp.zeros((16,), jnp.float32)
                f1 = jnp.zeros((16,), jnp.float32)
                for c in range(8):
                    w = wxy[c & 3] * wz[(c >> 2) & 1]
                    f0 = f0 + w * val0_v[pl.ds(c * _C + po, 16)]
                    f1 = f1 + w * val1_v[pl.ds(c * _C + po, 16)]
                feat_v[2 * l, pl.ds(po, 16)] = f0
                feat_v[2 * l + 1, pl.ds(po, 16)] = f1

        bufs = ((idx_a, val0_a, val1_a, sem_a), (idx_b, val0_b, val1_b, sem_b))

        def start(l, cb):
            idx_v, v0, v1, sem = bufs[l % 2]
            idx_pass(l, cb, idx_v)
            res, size, off, dense = _LEVELS[l]
            src0, src1 = (sh0, sh1) if off + size <= _SH_ROWS else (tab0_hbm, tab1_hbm)
            c0 = pltpu.async_copy(src0.at[idx_v], v0, sem)
            c1 = pltpu.async_copy(src1.at[idx_v], v1, sem)
            return (c0, c1)

        @pl.loop(0, _PER_W, step=_C)
        def _chunk(cb):
            cps = start(0, cb)
            for l in range(1, _NUM_LEVELS):
                nxt = start(l, cb)
                cps[0].wait()
                cps[1].wait()
                acc_pass(l - 1, cb, bufs[(l - 1) % 2][1], bufs[(l - 1) % 2][2])
                cps = nxt
            cps[0].wait()
            cps[1].wait()
            acc_pass(_NUM_LEVELS - 1, cb, bufs[(_NUM_LEVELS - 1) % 2][1],
                     bufs[(_NUM_LEVELS - 1) % 2][2])

            pltpu.sync_copy(feat_v, out_hbm.at[:, pl.ds(wbase + cb, _C)])

    return enc(xs, ys, zs, tab0, tab1)


def _mlp(feat_t, W0, b0, W1, b1, Wout, bout):
    B = 4096

    def body(x_ref, w0, b0r, w1, b1r, wo, bor, sig_ref, geo_ref):
        x = x_ref[...]
        h = jax.lax.dot_general(w0[...], x, (((0,), (0,)), ((), ())),
                                preferred_element_type=jnp.float32)
        h = jnp.maximum(h + b0r[...], 0.0)
        h = jax.lax.dot_general(w1[...], h, (((0,), (0,)), ((), ())),
                                preferred_element_type=jnp.float32)
        h = jnp.maximum(h + b1r[...], 0.0)
        o = jax.lax.dot_general(wo[...], h, (((0,), (0,)), ((), ())),
                                preferred_element_type=jnp.float32)
        o = o + bor[...]
        sig_ref[...] = jnp.exp(jnp.clip(o[:1, :], -15.0, 15.0))
        geo_ref[...] = o[1:, :]

    sig, geo_t = pl.pallas_call(
        body,
        grid=(_N // B,),
        in_specs=[
            pl.BlockSpec((_IN_DIM, B), lambda i: (0, i)),
            pl.BlockSpec((_IN_DIM, 64), lambda i: (0, 0)),
            pl.BlockSpec((64, 1), lambda i: (0, 0)),
            pl.BlockSpec((64, 64), lambda i: (0, 0)),
            pl.BlockSpec((64, 1), lambda i: (0, 0)),
            pl.BlockSpec((64, 16), lambda i: (0, 0)),
            pl.BlockSpec((16, 1), lambda i: (0, 0)),
        ],
        out_specs=[
            pl.BlockSpec((1, B), lambda i: (0, i)),
            pl.BlockSpec((15, B), lambda i: (0, i)),
        ],
        out_shape=[
            jax.ShapeDtypeStruct((1, _N), jnp.float32),
            jax.ShapeDtypeStruct((15, _N), jnp.float32),
        ],
    )(feat_t, W0, b0.reshape(-1, 1), W1, b1.reshape(-1, 1), Wout, bout.reshape(-1, 1))
    return sig.reshape(-1), geo_t.T


def kernel(xyzs, table, W0, b0, W1, b1, Wout, bout):
    feat_t = _encode(xyzs[:, 0], xyzs[:, 1], xyzs[:, 2], table[:, 0], table[:, 1])
    return _mlp(feat_t, W0, b0, W1, b1, Wout, bout)
